# baseline (speedup 1.0000x reference)
import functools

import jax
import jax.numpy as jnp
from jax import lax
from jax.experimental import pallas as pl
from jax.experimental.pallas import tpu as pltpu
from jax.experimental.pallas import tpu_sc as plsc

NN = 20000
EE = 320000
F = 16
F2 = 32
HID = 128

NSPLIT = 2
EH = EE // NSPLIT
NC = 2
NS = 16
NW = NC * NS
EPW = EH // NW
CHUNK = 40
NCH = EPW // CHUNK
NBUF = 5
NGRP = NCH // NBUF
STRIPE = 1280
NP = STRIPE * NS

_SC_PARAMS = pltpu.CompilerParams(use_tc_tiling_on_sc=False)


def _sc_mesh():
    return plsc.VectorSubcoreMesh(core_axis_name="c", subcore_axis_name="s",
                                  num_cores=NC, num_subcores=NS)


@functools.cache
def _build_sc_gather():
    @functools.partial(
        pl.kernel,
        out_type=jax.ShapeDtypeStruct((EH, 128), jnp.float32),
        mesh=_sc_mesh(),
        scratch_types=[
            pltpu.VMEM((NCH, CHUNK), jnp.int32),
            pltpu.VMEM((NBUF, CHUNK, F), jnp.float32),
            pltpu.SemaphoreType.DMA,
            pltpu.SemaphoreType.DMA,
        ],
        compiler_params=_SC_PARAMS,
    )
    def _sc_gather(x_hbm, src_hbm, out_hbm, idx_v, buf_v, gsem, wsem):
        cid = lax.axis_index("c")
        sid = lax.axis_index("s")
        wid = sid * NC + cid
        pltpu.sync_copy(src_hbm.at[wid], idx_v)

        def group(g, carry):
            j0 = g * NBUF
            for b in range(NBUF):
                pltpu.async_copy(x_hbm.at[idx_v.at[j0 + b]], buf_v.at[b],
                                 gsem)
            for b in range(NBUF):
                pltpu.make_async_copy(x_hbm.at[idx_v.at[j0 + b]],
                                      buf_v.at[b], gsem).wait()
                pltpu.async_copy(
                    buf_v.at[b],
                    out_hbm.at[pl.ds(wid * EPW + (j0 + b) * CHUNK, CHUNK),
                               pl.ds(0, F)], wsem)
            for b in range(NBUF):
                pltpu.make_async_copy(
                    buf_v.at[b],
                    out_hbm.at[pl.ds(wid * EPW + (j0 + b) * CHUNK, CHUNK),
                               pl.ds(0, F)], wsem).wait()
            return carry

        lax.fori_loop(0, NGRP, group, 0, unroll=False)

    return _sc_gather


@functools.cache
def _build_sc_scatter():
    @functools.partial(
        pl.kernel,
        out_type=jax.ShapeDtypeStruct((NC, NP, 128), jnp.float32),
        mesh=_sc_mesh(),
        scratch_types=[
            pltpu.VMEM((NCH, CHUNK), jnp.int32),
            pltpu.VMEM((NBUF, CHUNK, F2), jnp.float32),
            pltpu.VMEM_SHARED((NP, F2), jnp.float32),
            pltpu.SemaphoreType.DMA,
        ],
        compiler_params=_SC_PARAMS,
    )
    def _sc_scatter(msg_hbm, dst_hbm, zeros_hbm, acc_out,
                    idx_v, mbuf, acc_sh, msem):
        cid = lax.axis_index("c")
        sid = lax.axis_index("s")
        wid = sid * NC + cid
        row0 = sid * STRIPE
        pltpu.sync_copy(zeros_hbm, acc_sh.at[pl.ds(row0, STRIPE)])
        pltpu.sync_copy(dst_hbm.at[wid], idx_v)
        plsc.subcore_barrier()

        def group(g, carry):
            j0 = g * NBUF
            for b in range(NBUF):
                pltpu.async_copy(
                    msg_hbm.at[pl.ds(wid * EPW + (j0 + b) * CHUNK, CHUNK),
                               pl.ds(0, F2)], mbuf.at[b], msem)
            for b in range(NBUF):
                pltpu.make_async_copy(
                    msg_hbm.at[pl.ds(wid * EPW + (j0 + b) * CHUNK, CHUNK),
                               pl.ds(0, F2)], mbuf.at[b], msem).wait()
                pltpu.sync_copy(mbuf.at[b], acc_sh.at[idx_v.at[j0 + b]],
                                add=True)
            return carry

        lax.fori_loop(0, NGRP, group, 0, unroll=False)
        plsc.subcore_barrier()
        pltpu.sync_copy(acc_sh.at[pl.ds(row0, STRIPE)],
                        acc_out.at[cid, pl.ds(row0, STRIPE), pl.ds(0, F2)])

    return _sc_scatter


BE = 1280


def _gelu(z):
    return 0.5 * z * (1.0 + lax.erf(z * 0.7071067811865476))


def _edge_body(ea_ref, xj_ref, Wv4_ref, bv_ref, W1ea_ref, W1s_ref, b1_ref,
               g_ref, bb_ref, W2_ref, b2_ref, P_ref, G_ref, J_ref, R_ref,
               msg_ref):
    dot = functools.partial(jnp.dot, preferred_element_type=jnp.float32)
    ea = jnp.transpose(ea_ref[...])
    se = _gelu(dot(ea, Wv4_ref[...]) + bv_ref[...])
    h = dot(ea, W1ea_ref[...]) + dot(se, W1s_ref[...]) + b1_ref[...]
    mu = dot(h, J_ref[...])
    d = h - mu
    var = dot(d * d, J_ref[...])
    hn = d * lax.rsqrt(var + 1e-5) * g_ref[...] + bb_ref[...]
    h2 = _gelu(hn)
    w = dot(h2, W2_ref[...]) + b2_ref[...]
    n2 = dot(w * w, R_ref[...])
    inv = lax.rsqrt(jnp.maximum(n2, 1e-12))
    xrep = dot(xj_ref[:, 0:F], P_ref[...])
    msg_ref[:, 0:F] = dot(w * xrep, G_ref[...]) * inv
    msg_ref[:, F:F2] = jnp.ones((BE, F), jnp.float32)


def _edge_pipeline(half, eaT, xj, Wv, bv, W1, b1, ln_g, ln_b, W2, b2):
    P = jnp.repeat(jnp.eye(F, dtype=jnp.float32), F, axis=1)
    G = jnp.tile(jnp.eye(F, dtype=jnp.float32), (F, 1))
    J = jnp.full((HID, HID), 1.0 / HID, jnp.float32)
    R = jnp.ones((F * F, F), jnp.float32)
    Wv4 = jnp.concatenate([jnp.zeros((1, F), jnp.float32), Wv], axis=0)
    W1ea = jnp.concatenate([W1[0:1, :], jnp.zeros((3, HID), jnp.float32)],
                           axis=0)
    full = lambda shape: pl.BlockSpec(shape, lambda i: (0, 0))
    off = half * (EH // BE)
    return pl.pallas_call(
        _edge_body,
        grid=(EH // BE,),
        in_specs=[
            pl.BlockSpec((4, BE), lambda i: (0, i + off)),
            pl.BlockSpec((BE, 128), lambda i: (i, 0)),
            full((4, F)),
            full((1, F)),
            full((4, HID)),
            full((F, HID)),
            full((1, HID)),
            full((1, HID)),
            full((1, HID)),
            full((HID, F * F)),
            full((1, F * F)),
            full((F, F * F)),
            full((F * F, F)),
            full((HID, HID)),
            full((F * F, F)),
        ],
        out_specs=pl.BlockSpec((BE, 128), lambda i: (i, 0)),
        out_shape=jax.ShapeDtypeStruct((EH, 128), jnp.float32),
        name=f"edge_mlp_h{half}",
    )(eaT, xj, Wv4, bv.reshape(1, F), W1ea, W1[1:, :],
      b1.reshape(1, HID), ln_g.reshape(1, HID), ln_b.reshape(1, HID),
      W2, b2.reshape(1, F * F), P, G, J, R)


BN = 1000


def _final_body(acc_a_ref, acc_b_ref, x_ref, WrT_ref, bias_ref, out_ref):
    s = (acc_a_ref[0, :, 0:F] + acc_a_ref[1, :, 0:F]
         + acc_b_ref[0, :, 0:F] + acc_b_ref[1, :, 0:F])
    c = (acc_a_ref[0, :, F:F2] + acc_a_ref[1, :, F:F2]
         + acc_b_ref[0, :, F:F2] + acc_b_ref[1, :, F:F2])
    agg = jnp.where(c > 0, s / jnp.where(c > 0, c, 1.0), 0.0)
    root = (jnp.dot(x_ref[...], WrT_ref[...],
                    preferred_element_type=jnp.float32) + bias_ref[...])
    out_ref[...] = _gelu(agg + root)


def _finalize(acc_a, acc_b, x, WrT, bias):
    full = lambda shape: pl.BlockSpec(shape, lambda i: (0, 0))
    accblk = pl.BlockSpec((NC, BN, 128), lambda i: (0, i, 0))
    blk = pl.BlockSpec((BN, F), lambda i: (i, 0))
    return pl.pallas_call(
        _final_body,
        grid=(NN // BN,),
        in_specs=[accblk, accblk, blk, full((F, F)), full((1, F))],
        out_specs=blk,
        out_shape=jax.ShapeDtypeStruct((NN, F), jnp.float32),
    )(acc_a, acc_b, x, WrT, bias.reshape(1, F))


def kernel(x, edge_attr, Wv, bv, W1, b1, ln_g, ln_b, W2, b2, Wr, bias,
           edge_index):
    eaT = edge_attr.T
    src = edge_index[0].reshape(NSPLIT, NW, NCH, CHUNK)
    dst = edge_index[1].reshape(NSPLIT, NW, NCH, CHUNK)
    zeros = jnp.zeros((STRIPE, F2), jnp.float32)
    gather = _build_sc_gather()
    scatter = _build_sc_scatter()

    xj0 = gather(x, src[0])
    xj1 = gather(x, src[1])
    msg0 = _edge_pipeline(0, eaT, xj0, Wv, bv, W1, b1, ln_g, ln_b, W2, b2)
    msg1 = _edge_pipeline(1, eaT, xj1, Wv, bv, W1, b1, ln_g, ln_b, W2, b2)
    acc0 = scatter(msg0, dst[0], zeros)
    acc1 = scatter(msg1, dst[1], zeros)
    return _finalize(acc0, acc1, x, Wr.T, bias)


# speedup vs baseline: 1.0674x; 1.0674x over previous
import functools

import jax
import jax.numpy as jnp
from jax import lax
from jax.experimental import pallas as pl
from jax.experimental.pallas import tpu as pltpu
from jax.experimental.pallas import tpu_sc as plsc

NN = 20000
EE = 320000
F = 16
F2 = 32
HID = 128

NC = 2
NS = 16
NW = NC * NS
CHUNK = 80
NBUF = 5
NCH0 = 65
NCH1 = 60
EH0 = NW * NCH0 * CHUNK
EH1 = NW * NCH1 * CHUNK
STRIPE = 1280
NP = STRIPE * NS

_SC_PARAMS = pltpu.CompilerParams(use_tc_tiling_on_sc=False)


def _sc_mesh():
    return plsc.VectorSubcoreMesh(core_axis_name="c", subcore_axis_name="s",
                                  num_cores=NC, num_subcores=NS)


@functools.cache
def _build_sc_gather(nch):
    epw = nch * CHUNK
    ngrp = nch // NBUF

    @functools.partial(
        pl.kernel,
        out_type=jax.ShapeDtypeStruct((NW * epw, 128), jnp.float32),
        mesh=_sc_mesh(),
        scratch_types=[
            pltpu.VMEM((nch, CHUNK), jnp.int32),
            pltpu.VMEM((NBUF, CHUNK, F), jnp.float32),
            pltpu.SemaphoreType.DMA,
            pltpu.SemaphoreType.DMA,
        ],
        compiler_params=_SC_PARAMS,
    )
    def _sc_gather(x_hbm, src_hbm, out_hbm, idx_v, buf_v, gsem, wsem):
        cid = lax.axis_index("c")
        sid = lax.axis_index("s")
        wid = sid * NC + cid
        pltpu.sync_copy(src_hbm.at[wid], idx_v)

        def group(g, carry):
            j0 = g * NBUF
            for b in range(NBUF):
                pltpu.async_copy(x_hbm.at[idx_v.at[j0 + b]], buf_v.at[b],
                                 gsem)
            for b in range(NBUF):
                pltpu.make_async_copy(x_hbm.at[idx_v.at[j0 + b]],
                                      buf_v.at[b], gsem).wait()
                pltpu.async_copy(
                    buf_v.at[b],
                    out_hbm.at[pl.ds(wid * epw + (j0 + b) * CHUNK, CHUNK),
                               pl.ds(0, F)], wsem)
            for b in range(NBUF):
                pltpu.make_async_copy(
                    buf_v.at[b],
                    out_hbm.at[pl.ds(wid * epw + (j0 + b) * CHUNK, CHUNK),
                               pl.ds(0, F)], wsem).wait()
            return carry

        lax.fori_loop(0, ngrp, group, 0, unroll=False)

    return _sc_gather


@functools.cache
def _build_sc_scatter(nch, chained):
    epw = nch * CHUNK
    ngrp = nch // NBUF

    @functools.partial(
        pl.kernel,
        out_type=jax.ShapeDtypeStruct((NC, NP, 128), jnp.float32),
        mesh=_sc_mesh(),
        scratch_types=[
            pltpu.VMEM((nch, CHUNK), jnp.int32),
            pltpu.VMEM((NBUF, CHUNK, F2), jnp.float32),
            pltpu.VMEM_SHARED((NP, F2), jnp.float32),
            pltpu.SemaphoreType.DMA,
        ],
        compiler_params=_SC_PARAMS,
    )
    def _sc_scatter(msg_hbm, dst_hbm, init_hbm, acc_out,
                    idx_v, mbuf, acc_sh, msem):
        cid = lax.axis_index("c")
        sid = lax.axis_index("s")
        wid = sid * NC + cid
        row0 = sid * STRIPE
        if chained:
            pltpu.sync_copy(
                init_hbm.at[cid, pl.ds(row0, STRIPE), pl.ds(0, F2)],
                acc_sh.at[pl.ds(row0, STRIPE)])
        else:
            pltpu.sync_copy(init_hbm, acc_sh.at[pl.ds(row0, STRIPE)])
        pltpu.sync_copy(dst_hbm.at[wid], idx_v)
        plsc.subcore_barrier()

        def group(g, carry):
            j0 = g * NBUF
            for b in range(NBUF):
                pltpu.async_copy(
                    msg_hbm.at[pl.ds(wid * epw + (j0 + b) * CHUNK, CHUNK),
                               pl.ds(0, F2)], mbuf.at[b], msem)
            for b in range(NBUF):
                pltpu.make_async_copy(
                    msg_hbm.at[pl.ds(wid * epw + (j0 + b) * CHUNK, CHUNK),
                               pl.ds(0, F2)], mbuf.at[b], msem).wait()
                pltpu.sync_copy(mbuf.at[b], acc_sh.at[idx_v.at[j0 + b]],
                                add=True)
            return carry

        lax.fori_loop(0, ngrp, group, 0, unroll=False)
        plsc.subcore_barrier()
        pltpu.sync_copy(acc_sh.at[pl.ds(row0, STRIPE)],
                        acc_out.at[cid, pl.ds(row0, STRIPE), pl.ds(0, F2)])

    return _sc_scatter


BE = 1280


def _gelu(z):
    return 0.5 * z * (1.0 + lax.erf(z * 0.7071067811865476))


def _edge_body(ea_ref, xj_ref, Wv4_ref, bv_ref, W1ea_ref, W1s_ref, b1_ref,
               g_ref, bb_ref, W2_ref, b2_ref, P_ref, G_ref, J_ref, R_ref,
               msg_ref):
    dot = functools.partial(jnp.dot, preferred_element_type=jnp.float32)
    ea = jnp.transpose(ea_ref[...])
    se = _gelu(dot(ea, Wv4_ref[...]) + bv_ref[...])
    h = dot(ea, W1ea_ref[...]) + dot(se, W1s_ref[...]) + b1_ref[...]
    mu = dot(h, J_ref[...])
    d = h - mu
    var = dot(d * d, J_ref[...])
    hn = d * lax.rsqrt(var + 1e-5) * g_ref[...] + bb_ref[...]
    h2 = _gelu(hn)
    w = dot(h2, W2_ref[...]) + b2_ref[...]
    n2 = dot(w * w, R_ref[...])
    inv = lax.rsqrt(jnp.maximum(n2, 1e-12))
    xrep = dot(xj_ref[:, 0:F], P_ref[...])
    msg_ref[:, 0:F] = dot(w * xrep, G_ref[...]) * inv
    msg_ref[:, F:F2] = jnp.ones((BE, F), jnp.float32)


def _edge_pipeline(name, blk_off, eh, eaT, xj, Wv, bv, W1, b1, ln_g, ln_b,
                   W2, b2):
    P = jnp.repeat(jnp.eye(F, dtype=jnp.float32), F, axis=1)
    G = jnp.tile(jnp.eye(F, dtype=jnp.float32), (F, 1))
    J = jnp.full((HID, HID), 1.0 / HID, jnp.float32)
    R = jnp.ones((F * F, F), jnp.float32)
    Wv4 = jnp.concatenate([jnp.zeros((1, F), jnp.float32), Wv], axis=0)
    W1ea = jnp.concatenate([W1[0:1, :], jnp.zeros((3, HID), jnp.float32)],
                           axis=0)
    full = lambda shape: pl.BlockSpec(shape, lambda i: (0, 0))
    return pl.pallas_call(
        _edge_body,
        grid=(eh // BE,),
        in_specs=[
            pl.BlockSpec((4, BE), lambda i: (0, i + blk_off)),
            pl.BlockSpec((BE, 128), lambda i: (i, 0)),
            full((4, F)),
            full((1, F)),
            full((4, HID)),
            full((F, HID)),
            full((1, HID)),
            full((1, HID)),
            full((1, HID)),
            full((HID, F * F)),
            full((1, F * F)),
            full((F, F * F)),
            full((F * F, F)),
            full((HID, HID)),
            full((F * F, F)),
        ],
        out_specs=pl.BlockSpec((BE, 128), lambda i: (i, 0)),
        out_shape=jax.ShapeDtypeStruct((eh, 128), jnp.float32),
        name=name,
    )(eaT, xj, Wv4, bv.reshape(1, F), W1ea, W1[1:, :],
      b1.reshape(1, HID), ln_g.reshape(1, HID), ln_b.reshape(1, HID),
      W2, b2.reshape(1, F * F), P, G, J, R)


BN = 1000


def _final_body(acc_ref, x_ref, WrT_ref, bias_ref, out_ref):
    s = acc_ref[0, :, 0:F] + acc_ref[1, :, 0:F]
    c = acc_ref[0, :, F:F2] + acc_ref[1, :, F:F2]
    agg = jnp.where(c > 0, s / jnp.where(c > 0, c, 1.0), 0.0)
    root = (jnp.dot(x_ref[...], WrT_ref[...],
                    preferred_element_type=jnp.float32) + bias_ref[...])
    out_ref[...] = _gelu(agg + root)


def _finalize(acc, x, WrT, bias):
    full = lambda shape: pl.BlockSpec(shape, lambda i: (0, 0))
    accblk = pl.BlockSpec((NC, BN, 128), lambda i: (0, i, 0))
    blk = pl.BlockSpec((BN, F), lambda i: (i, 0))
    return pl.pallas_call(
        _final_body,
        grid=(NN // BN,),
        in_specs=[accblk, blk, full((F, F)), full((1, F))],
        out_specs=blk,
        out_shape=jax.ShapeDtypeStruct((NN, F), jnp.float32),
    )(acc, x, WrT, bias.reshape(1, F))


def kernel(x, edge_attr, Wv, bv, W1, b1, ln_g, ln_b, W2, b2, Wr, bias,
           edge_index):
    eaT = edge_attr.T
    src0 = edge_index[0, :EH0].reshape(NW, NCH0, CHUNK)
    src1 = edge_index[0, EH0:].reshape(NW, NCH1, CHUNK)
    dst0 = edge_index[1, :EH0].reshape(NW, NCH0, CHUNK)
    dst1 = edge_index[1, EH0:].reshape(NW, NCH1, CHUNK)
    zeros = jnp.zeros((STRIPE, F2), jnp.float32)

    xj0 = _build_sc_gather(NCH0)(x, src0)
    xj1 = _build_sc_gather(NCH1)(x, src1)
    msg0 = _edge_pipeline("edge_mlp_h0", 0, EH0, eaT, xj0, Wv, bv, W1, b1,
                          ln_g, ln_b, W2, b2)
    msg1 = _edge_pipeline("edge_mlp_h1", EH0 // BE, EH1, eaT, xj1, Wv, bv,
                          W1, b1, ln_g, ln_b, W2, b2)
    acc0 = _build_sc_scatter(NCH0, False)(msg0, dst0, zeros)
    acc = _build_sc_scatter(NCH1, True)(msg1, dst1, acc0)
    return _finalize(acc, x, Wr.T, bias)


# speedup vs baseline: 1.0744x; 1.0066x over previous
import functools

import jax
import jax.numpy as jnp
from jax import lax
from jax.experimental import pallas as pl
from jax.experimental.pallas import tpu as pltpu
from jax.experimental.pallas import tpu_sc as plsc

NN = 20000
EE = 320000
F = 16
F2 = 32
HID = 128

NC = 2
NS = 16
NW = NC * NS
CHUNK = 80
NBUF = 5
NCH0 = 65
NCH1 = 60
EH0 = NW * NCH0 * CHUNK
EH1 = NW * NCH1 * CHUNK
STRIPE = 1280
NP = STRIPE * NS

_SC_PARAMS = pltpu.CompilerParams(use_tc_tiling_on_sc=False)


def _sc_mesh():
    return plsc.VectorSubcoreMesh(core_axis_name="c", subcore_axis_name="s",
                                  num_cores=NC, num_subcores=NS)


@functools.cache
def _build_sc_gather(nch):
    epw = nch * CHUNK
    ngrp = nch // NBUF

    @functools.partial(
        pl.kernel,
        out_type=jax.ShapeDtypeStruct((NW * epw, 128), jnp.float32),
        mesh=_sc_mesh(),
        scratch_types=[
            pltpu.VMEM((nch, CHUNK), jnp.int32),
            pltpu.VMEM((NBUF, CHUNK, F), jnp.float32),
            pltpu.SemaphoreType.DMA,
            pltpu.SemaphoreType.DMA,
        ],
        compiler_params=_SC_PARAMS,
    )
    def _sc_gather(x_hbm, src_hbm, out_hbm, idx_v, buf_v, gsem, wsem):
        cid = lax.axis_index("c")
        sid = lax.axis_index("s")
        wid = sid * NC + cid
        pltpu.sync_copy(src_hbm.at[wid], idx_v)

        def group(g, carry):
            j0 = g * NBUF
            for b in range(NBUF):
                pltpu.async_copy(x_hbm.at[idx_v.at[j0 + b]], buf_v.at[b],
                                 gsem)
            for b in range(NBUF):
                pltpu.make_async_copy(x_hbm.at[idx_v.at[j0 + b]],
                                      buf_v.at[b], gsem).wait()
                pltpu.async_copy(
                    buf_v.at[b],
                    out_hbm.at[pl.ds(wid * epw + (j0 + b) * CHUNK, CHUNK),
                               pl.ds(0, F)], wsem)
            for b in range(NBUF):
                pltpu.make_async_copy(
                    buf_v.at[b],
                    out_hbm.at[pl.ds(wid * epw + (j0 + b) * CHUNK, CHUNK),
                               pl.ds(0, F)], wsem).wait()
            return carry

        lax.fori_loop(0, ngrp, group, 0, unroll=False)

    return _sc_gather


@functools.cache
def _build_sc_scatter(nch, chained):
    epw = nch * CHUNK
    ngrp = nch // NBUF

    @functools.partial(
        pl.kernel,
        out_type=jax.ShapeDtypeStruct((NC, NP, 128), jnp.float32),
        mesh=_sc_mesh(),
        scratch_types=[
            pltpu.VMEM((nch, CHUNK), jnp.int32),
            pltpu.VMEM((NBUF, CHUNK, F2), jnp.float32),
            pltpu.VMEM_SHARED((NP, F2), jnp.float32),
            pltpu.SemaphoreType.DMA,
        ],
        compiler_params=_SC_PARAMS,
    )
    def _sc_scatter(msg_hbm, dst_hbm, init_hbm, acc_out,
                    idx_v, mbuf, acc_sh, msem):
        cid = lax.axis_index("c")
        sid = lax.axis_index("s")
        wid = sid * NC + cid
        row0 = sid * STRIPE
        if chained:
            pltpu.sync_copy(
                init_hbm.at[cid, pl.ds(row0, STRIPE), pl.ds(0, F2)],
                acc_sh.at[pl.ds(row0, STRIPE)])
        else:
            pltpu.sync_copy(init_hbm, acc_sh.at[pl.ds(row0, STRIPE)])
        pltpu.sync_copy(dst_hbm.at[wid], idx_v)
        plsc.subcore_barrier()

        def group(g, carry):
            j0 = g * NBUF
            for b in range(NBUF):
                pltpu.async_copy(
                    msg_hbm.at[pl.ds(wid * epw + (j0 + b) * CHUNK, CHUNK),
                               pl.ds(0, F2)], mbuf.at[b], msem)
            for b in range(NBUF):
                pltpu.make_async_copy(
                    msg_hbm.at[pl.ds(wid * epw + (j0 + b) * CHUNK, CHUNK),
                               pl.ds(0, F2)], mbuf.at[b], msem).wait()
                pltpu.sync_copy(mbuf.at[b], acc_sh.at[idx_v.at[j0 + b]],
                                add=True)
            return carry

        lax.fori_loop(0, ngrp, group, 0, unroll=False)
        plsc.subcore_barrier()
        pltpu.sync_copy(acc_sh.at[pl.ds(row0, STRIPE)],
                        acc_out.at[cid, pl.ds(row0, STRIPE), pl.ds(0, F2)])

    return _sc_scatter


BE = 1280


def _gelu(z):
    return 0.5 * z * (1.0 + lax.erf(z * 0.7071067811865476))


def _edge_body(ea_ref, xj_ref, Wv4_ref, bv_ref, W1ea_ref, W1s_ref, b1_ref,
               g_ref, bb_ref, W2_ref, b2_ref, P_ref, G_ref, J_ref, R_ref,
               msg_ref):
    dot = functools.partial(jnp.dot, preferred_element_type=jnp.float32)
    ea = jnp.transpose(ea_ref[...])
    se = _gelu(dot(ea, Wv4_ref[...]) + bv_ref[...])
    h = dot(ea, W1ea_ref[...]) + dot(se, W1s_ref[...]) + b1_ref[...]
    mu = dot(h, J_ref[...])
    d = h - mu
    var = dot(d * d, J_ref[...])
    hn = d * lax.rsqrt(var + 1e-5) * g_ref[...] + bb_ref[...]
    h2 = _gelu(hn)
    w = dot(h2, W2_ref[...]) + b2_ref[...]
    n2 = dot(w * w, R_ref[...])
    inv = lax.rsqrt(jnp.maximum(n2, 1e-12))
    xrep = dot(xj_ref[:, 0:F], P_ref[...])
    msg_ref[:, 0:F] = dot(w * xrep, G_ref[...]) * inv
    msg_ref[:, F:F2] = jnp.ones((BE, F), jnp.float32)


def _edge_pipeline(name, blk_off, eh, eaT, xj, Wv, bv, W1, b1, ln_g, ln_b,
                   W2, b2):
    P = jnp.repeat(jnp.eye(F, dtype=jnp.float32), F, axis=1)
    G = jnp.tile(jnp.eye(F, dtype=jnp.float32), (F, 1))
    J = jnp.full((HID, HID), 1.0 / HID, jnp.float32)
    R = jnp.ones((F * F, F), jnp.float32)
    Wv4 = jnp.concatenate([jnp.zeros((1, F), jnp.float32), Wv], axis=0)
    W1ea = jnp.concatenate([W1[0:1, :], jnp.zeros((3, HID), jnp.float32)],
                           axis=0)
    full = lambda shape: pl.BlockSpec(shape, lambda i: (0, 0))
    return pl.pallas_call(
        _edge_body,
        grid=(eh // BE,),
        in_specs=[
            pl.BlockSpec((4, BE), lambda i: (0, i + blk_off)),
            pl.BlockSpec((BE, 128), lambda i: (i, 0)),
            full((4, F)),
            full((1, F)),
            full((4, HID)),
            full((F, HID)),
            full((1, HID)),
            full((1, HID)),
            full((1, HID)),
            full((HID, F * F)),
            full((1, F * F)),
            full((F, F * F)),
            full((F * F, F)),
            full((HID, HID)),
            full((F * F, F)),
        ],
        out_specs=pl.BlockSpec((BE, 128), lambda i: (i, 0)),
        out_shape=jax.ShapeDtypeStruct((eh, 128), jnp.float32),
        name=name,
    )(eaT, xj, Wv4, bv.reshape(1, F), W1ea, W1[1:, :],
      b1.reshape(1, HID), ln_g.reshape(1, HID), ln_b.reshape(1, HID),
      W2, b2.reshape(1, F * F), P, G, J, R)


BN = 2000


def _final_body(acc_ref, x_ref, WrT_ref, bias_ref, out_ref):
    s = acc_ref[0, :, 0:F] + acc_ref[1, :, 0:F]
    c = acc_ref[0, :, F:F2] + acc_ref[1, :, F:F2]
    agg = jnp.where(c > 0, s / jnp.where(c > 0, c, 1.0), 0.0)
    root = (jnp.dot(x_ref[...], WrT_ref[...],
                    preferred_element_type=jnp.float32) + bias_ref[...])
    out_ref[...] = _gelu(agg + root)


def _finalize(acc, x, WrT, bias):
    full = lambda shape: pl.BlockSpec(shape, lambda i: (0, 0))
    accblk = pl.BlockSpec((NC, BN, 128), lambda i: (0, i, 0))
    blk = pl.BlockSpec((BN, F), lambda i: (i, 0))
    return pl.pallas_call(
        _final_body,
        grid=(NN // BN,),
        in_specs=[accblk, blk, full((F, F)), full((1, F))],
        out_specs=blk,
        out_shape=jax.ShapeDtypeStruct((NN, F), jnp.float32),
    )(acc, x, WrT, bias.reshape(1, F))


def kernel(x, edge_attr, Wv, bv, W1, b1, ln_g, ln_b, W2, b2, Wr, bias,
           edge_index):
    eaT = edge_attr.T
    src0 = edge_index[0, :EH0].reshape(NW, NCH0, CHUNK)
    src1 = edge_index[0, EH0:].reshape(NW, NCH1, CHUNK)
    dst0 = edge_index[1, :EH0].reshape(NW, NCH0, CHUNK)
    dst1 = edge_index[1, EH0:].reshape(NW, NCH1, CHUNK)
    zeros = jnp.zeros((STRIPE, F2), jnp.float32)

    xj0 = _build_sc_gather(NCH0)(x, src0)
    xj1 = _build_sc_gather(NCH1)(x, src1)
    msg0 = _edge_pipeline("edge_mlp_h0", 0, EH0, eaT, xj0, Wv, bv, W1, b1,
                          ln_g, ln_b, W2, b2)
    msg1 = _edge_pipeline("edge_mlp_h1", EH0 // BE, EH1, eaT, xj1, Wv, bv,
                          W1, b1, ln_g, ln_b, W2, b2)
    acc0 = _build_sc_scatter(NCH0, False)(msg0, dst0, zeros)
    acc = _build_sc_scatter(NCH1, True)(msg1, dst1, acc0)
    return _finalize(acc, x, Wr.T, bias)
